# speedup vs baseline: 2.0236x; 2.0236x over previous
import functools
import jax
import jax.numpy as jnp
from jax import lax
from jax.experimental import pallas as pl
from jax.experimental.pallas import tpu as pltpu
from jax.experimental.pallas import tpu_sc as plsc

D = 128
EPS = 1e-12
NC = 2
NS = 16
NW = NC * NS
CHUNK = 128


def _sc_gather_body(idx_hbm, table_hbm, out_hbm, idx_v, buf0, buf1, sem0, sem1):
    wid = lax.axis_index("s") * NC + lax.axis_index("c")
    n_chunks = idx_hbm.shape[0] // NW
    rows_per_w = n_chunks * CHUNK
    pltpu.sync_copy(idx_hbm.at[pl.ds(wid * n_chunks, n_chunks)], idx_v)

    bufs = (buf0, buf1)
    sems = (sem0, sem1)

    def start(j, b):
        return pltpu.async_copy(table_hbm.at[idx_v.at[j]], bufs[b], sems[b])

    cp = start(0, 0)
    for j in range(n_chunks):
        b = j & 1
        nxt = start(j + 1, 1 - b) if j + 1 < n_chunks else None
        cp.wait()
        pltpu.sync_copy(bufs[b], out_hbm.at[pl.ds(wid * rows_per_w + j * CHUNK, CHUNK)])
        cp = nxt


def _sc_gather(idx2d, table):
    t_rows = idx2d.shape[0] * idx2d.shape[1]
    n_chunks = idx2d.shape[0] // NW
    fn = pl.kernel(
        _sc_gather_body,
        out_type=jax.ShapeDtypeStruct((t_rows, D), jnp.float32),
        mesh=plsc.VectorSubcoreMesh(core_axis_name="c", subcore_axis_name="s"),
        scratch_types=[
            pltpu.VMEM((n_chunks, CHUNK), jnp.int32),
            pltpu.VMEM((CHUNK, D), jnp.float32),
            pltpu.VMEM((CHUNK, D), jnp.float32),
            pltpu.SemaphoreType.DMA,
            pltpu.SemaphoreType.DMA,
        ],
    )
    return fn(idx2d, table)


def _ln_body(g_ref, tf_ref, tt_ref, pos_ref, gam_ref, bet_ref, o_ref):
    scale = jnp.sqrt(jnp.float32(D))
    g = g_ref[...]
    t = tf_ref[...]
    tt = tt_ref[...]
    te = tt[0:1, :] + t * (tt[1:2, :] - tt[0:1, :])
    x = scale * (g + te) + pos_ref[...]
    mean = jnp.mean(x, axis=-1, keepdims=True)
    xc = x - mean
    var = jnp.mean(xc * xc, axis=-1, keepdims=True)
    o_ref[...] = xc * lax.rsqrt(var + EPS) * gam_ref[...] + bet_ref[...]


def _ln_call(gathered, tf, type_table, pos_table, gamma, beta, seq_len):
    t_rows = gathered.shape[0]
    rows = 512
    n_pos_blocks = seq_len // rows
    grid = (t_rows // rows,)
    return pl.pallas_call(
        _ln_body,
        grid=grid,
        in_specs=[
            pl.BlockSpec((rows, D), lambda i: (i, 0)),
            pl.BlockSpec((rows, 1), lambda i: (i, 0)),
            pl.BlockSpec((2, D), lambda i: (0, 0)),
            pl.BlockSpec((rows, D), lambda i: (i % n_pos_blocks, 0)),
            pl.BlockSpec((1, D), lambda i: (0, 0)),
            pl.BlockSpec((1, D), lambda i: (0, 0)),
        ],
        out_specs=pl.BlockSpec((rows, D), lambda i: (i, 0)),
        out_shape=jax.ShapeDtypeStruct((t_rows, D), jnp.float32),
    )(gathered, tf, type_table, pos_table, gamma, beta)


def kernel(token_ids, type_ids, word_table, type_table, pos_table, ln_gamma, ln_beta):
    b, s = token_ids.shape
    t_rows = b * s
    idx2d = token_ids.astype(jnp.int32).reshape(t_rows // CHUNK, CHUNK)
    gathered = _sc_gather(idx2d, word_table)
    tf = type_ids.astype(jnp.float32).reshape(t_rows, 1)
    out2d = _ln_call(gathered, tf, type_table, pos_table,
                     ln_gamma.reshape(1, D), ln_beta.reshape(1, D), s)
    return out2d.reshape(b, s, D)
